# baseline (speedup 1.0000x reference)
import jax
import jax.numpy as jnp
from jax import lax
from jax.experimental import pallas as pl
from jax.experimental.pallas import tpu as pltpu

_B = 1024
_P = 100
_NF = 3
_NC = 2
_GNN = 30
_RD = 4
_K = 8
_G = 8
_PAD = 128


def _lrelu(a):
    return jnp.maximum(a, 0.2 * a)


def _dot(a, b):
    return jnp.dot(a, b, preferred_element_type=jnp.float32)


def _bdot(a, b, ca, cb):
    return lax.dot_general(
        a, b, (((ca,), (cb,)), ((0,), (0,))),
        preferred_element_type=jnp.float32)


def _body(x_ref, cond_ref, *refs):
    out_ref = refs[-1]
    (fW0, fb0, fW1, fb1, fW2, fb2,
     rW0, rb0, rW1, rb1,
     lW0, lb0, lW1, lb1,
     g1W0, g1b0, g1W1, g1b1,
     m1W0, m1b0, m1W1, m1b1,
     g2W0, g2b0, g2W1, g2b1,
     m2W0, m2b0, m2W1, m2b1,
     oW0, ob0, oW1, ob1) = [r[...] for r in refs[:-1]]

    x2 = x_ref[...].reshape(_G * _P, _NF)
    cond = cond_ref[...]

    f = _lrelu(_dot(x2, fW0) + fb0)
    f = _lrelu(_dot(f, fW1) + fb1)
    f = _lrelu(_dot(f, fW2) + fb2)
    pooled = jnp.max(f.reshape(_G, _P, 64), axis=1)
    r = _lrelu(_dot(pooled, rW0) + rb0)
    r = _dot(r, rW1) + rb1

    condb = jnp.broadcast_to(cond[:, None, :], (_G, _P, _NC))
    rb = jnp.broadcast_to(r[:, None, :], (_G, _P, _RD))
    cat = jnp.concatenate(
        [x_ref[...], condb, rb], axis=2).reshape(_G * _P, _NF + _NC + _RD)
    h = _lrelu(_dot(cat, lW0) + lb0)
    h = _dot(h, lW1) + lb1
    hb = h.reshape(_G, _P, _GNN)

    hbp = jnp.concatenate(
        [hb, jnp.zeros((_G, _PAD - _P, _GNN), jnp.float32)], axis=1)
    sq = jnp.sum(hb * hb, axis=-1)
    sqp = jnp.concatenate(
        [sq, jnp.full((_G, _PAD - _P), jnp.inf, jnp.float32)], axis=1)
    gram = _bdot(hb, hbp, 2, 2)
    d2 = sq[:, :, None] + sqp[:, None, :] - 2.0 * gram
    lane = lax.broadcasted_iota(
        jnp.int32, (_P, _PAD), 1).astype(jnp.float32)
    for _ in range(_K):
        m = jnp.min(d2, axis=2, keepdims=True)
        cand = jnp.where(d2 == m, lane, 128.0)
        jmin = jnp.min(cand, axis=2, keepdims=True)
        d2 = jnp.where(cand == jmin, jnp.inf, d2)
    adj = jnp.isinf(d2).astype(jnp.float32)

    aggr = _bdot(adj, hbp, 2, 1)
    z = (hb + aggr).reshape(_G * _P, _GNN)
    m = _lrelu(_dot(z, g1W0) + g1b0)
    m = _dot(m, g1W1) + g1b1
    mp = jnp.sum(m.reshape(_G, _P, _GNN), axis=1)
    pg = _dot(mp, m1W0[_GNN:]) + m1b0
    t = _dot(h, m1W0[:_GNN]).reshape(_G, _P, _GNN) + pg[:, None, :]
    h = _dot(_lrelu(t).reshape(_G * _P, _GNN), m1W1) + m1b1
    hb = h.reshape(_G, _P, _GNN)

    hbp = jnp.concatenate(
        [hb, jnp.zeros((_G, _PAD - _P, _GNN), jnp.float32)], axis=1)
    aggr = _bdot(adj, hbp, 2, 1)
    z = (hb + aggr).reshape(_G * _P, _GNN)
    m = _lrelu(_dot(z, g2W0) + g2b0)
    m = _dot(m, g2W1) + g2b1
    mp = jnp.sum(m.reshape(_G, _P, _GNN), axis=1)
    pg = _dot(mp, m2W0[_GNN:]) + m2b0
    t = _dot(h, m2W0[:_GNN]).reshape(_G, _P, _GNN) + pg[:, None, :]
    h = _dot(_lrelu(t).reshape(_G * _P, _GNN), m2W1) + m2b1

    ggn = jnp.sum(h.reshape(_G, _P, _GNN), axis=1)
    o = (_dot(r, oW0[:_RD]) + _dot(cond, oW0[_RD:_RD + _NC])
         + _dot(ggn, oW0[_RD + _NC:]) + ob0)
    o = _lrelu(o)
    o = _lrelu(_dot(o, oW1) + ob1)
    out_ref[...] = o


def kernel(x, cond, params, batch):
    del batch
    x3 = x.reshape(_B, _P, _NF)
    leaves = []
    for name in ('fc', 'rgan', 'lin', 'gin1', 'mpd1', 'gin2', 'mpd2', 'final'):
        for W, b in params[name]:
            leaves.append(W)
            leaves.append(b.reshape(1, -1))

    grid = (_B // _G,)
    in_specs = [
        pl.BlockSpec((_G, _P, _NF), lambda i: (i, 0, 0)),
        pl.BlockSpec((_G, _NC), lambda i: (i, 0)),
    ]
    for leaf in leaves:
        in_specs.append(pl.BlockSpec(leaf.shape, lambda i: (0,) * leaf.ndim))

    out = pl.pallas_call(
        _body,
        grid=grid,
        in_specs=in_specs,
        out_specs=pl.BlockSpec((_G, 1), lambda i: (i, 0)),
        out_shape=jax.ShapeDtypeStruct((_B, 1), jnp.float32),
        compiler_params=pltpu.CompilerParams(
            dimension_semantics=("parallel",)),
    )(x3, cond, *leaves)
    return out.reshape(_B)


# speedup vs baseline: 1.2955x; 1.2955x over previous
import jax
import jax.numpy as jnp
from jax import lax
from jax.experimental import pallas as pl
from jax.experimental.pallas import tpu as pltpu

_B = 1024
_P = 100
_NF = 3
_NC = 2
_GNN = 30
_RD = 4
_K = 8
_G = 16
_PAD = 128


def _lrelu(a):
    return jnp.maximum(a, 0.2 * a)


def _dot(a, b):
    return jnp.dot(a, b, preferred_element_type=jnp.float32)


def _bdot(a, b, ca, cb):
    return lax.dot_general(
        a, b, (((ca,), (cb,)), ((0,), (0,))),
        preferred_element_type=jnp.float32)


def _body(x_ref, cond_ref, *refs):
    out_ref = refs[-1]
    (fW0, fb0, fW1, fb1, fW2, fb2,
     rW0, rb0, rW1, rb1,
     lW0, lb0, lW1, lb1,
     g1W0, g1b0, g1W1, g1b1,
     m1W0, m1b0, m1W1, m1b1,
     g2W0, g2b0, g2W1, g2b1,
     m2W0, m2b0, m2W1, m2b1,
     oW0, ob0, oW1, ob1) = [r[...] for r in refs[:-1]]

    x2 = x_ref[...].reshape(_G * _P, _NF)
    cond = cond_ref[...]

    f = _lrelu(_dot(x2, fW0) + fb0)
    f = _lrelu(_dot(f, fW1) + fb1)
    f = _lrelu(_dot(f, fW2) + fb2)
    pooled = jnp.max(f.reshape(_G, _P, 64), axis=1)
    r = _lrelu(_dot(pooled, rW0) + rb0)
    r = _dot(r, rW1) + rb1

    condb = jnp.broadcast_to(cond[:, None, :], (_G, _P, _NC))
    rb = jnp.broadcast_to(r[:, None, :], (_G, _P, _RD))
    cat = jnp.concatenate(
        [x_ref[...], condb, rb], axis=2).reshape(_G * _P, _NF + _NC + _RD)
    h = _lrelu(_dot(cat, lW0) + lb0)
    h = _dot(h, lW1) + lb1
    hb = h.reshape(_G, _P, _GNN)

    hbp = jnp.concatenate(
        [hb, jnp.zeros((_G, _PAD - _P, _GNN), jnp.float32)], axis=1)
    sq = jnp.sum(hb * hb, axis=-1)
    sqp = jnp.concatenate(
        [sq, jnp.full((_G, _PAD - _P), jnp.inf, jnp.float32)], axis=1)
    gram = _bdot(hb, hbp, 2, 2)
    d2 = sq[:, :, None] + sqp[:, None, :] - 2.0 * gram
    lane = lax.broadcasted_iota(
        jnp.int32, (_P, _PAD), 1).astype(jnp.float32)
    for _ in range(_K):
        m = jnp.min(d2, axis=2, keepdims=True)
        cand = jnp.where(d2 == m, lane, 128.0)
        jmin = jnp.min(cand, axis=2, keepdims=True)
        d2 = jnp.where(cand == jmin, jnp.inf, d2)
    adj = jnp.isinf(d2).astype(jnp.float32)

    aggr = _bdot(adj, hbp, 2, 1)
    z = (hb + aggr).reshape(_G * _P, _GNN)
    m = _lrelu(_dot(z, g1W0) + g1b0)
    m = _dot(m, g1W1) + g1b1
    mp = jnp.sum(m.reshape(_G, _P, _GNN), axis=1)
    pg = _dot(mp, m1W0[_GNN:]) + m1b0
    t = _dot(h, m1W0[:_GNN]).reshape(_G, _P, _GNN) + pg[:, None, :]
    h = _dot(_lrelu(t).reshape(_G * _P, _GNN), m1W1) + m1b1
    hb = h.reshape(_G, _P, _GNN)

    hbp = jnp.concatenate(
        [hb, jnp.zeros((_G, _PAD - _P, _GNN), jnp.float32)], axis=1)
    aggr = _bdot(adj, hbp, 2, 1)
    z = (hb + aggr).reshape(_G * _P, _GNN)
    m = _lrelu(_dot(z, g2W0) + g2b0)
    m = _dot(m, g2W1) + g2b1
    mp = jnp.sum(m.reshape(_G, _P, _GNN), axis=1)
    pg = _dot(mp, m2W0[_GNN:]) + m2b0
    t = _dot(h, m2W0[:_GNN]).reshape(_G, _P, _GNN) + pg[:, None, :]
    h = _dot(_lrelu(t).reshape(_G * _P, _GNN), m2W1) + m2b1

    ggn = jnp.sum(h.reshape(_G, _P, _GNN), axis=1)
    o = (_dot(r, oW0[:_RD]) + _dot(cond, oW0[_RD:_RD + _NC])
         + _dot(ggn, oW0[_RD + _NC:]) + ob0)
    o = _lrelu(o)
    o = _lrelu(_dot(o, oW1) + ob1)
    out_ref[...] = o


def kernel(x, cond, params, batch):
    del batch
    x3 = x.reshape(_B, _P, _NF)
    leaves = []
    for name in ('fc', 'rgan', 'lin', 'gin1', 'mpd1', 'gin2', 'mpd2', 'final'):
        for W, b in params[name]:
            leaves.append(W)
            leaves.append(b.reshape(1, -1))

    grid = (_B // _G,)
    in_specs = [
        pl.BlockSpec((_G, _P, _NF), lambda i: (i, 0, 0)),
        pl.BlockSpec((_G, _NC), lambda i: (i, 0)),
    ]
    for leaf in leaves:
        in_specs.append(pl.BlockSpec(leaf.shape, lambda i: (0,) * leaf.ndim))

    out = pl.pallas_call(
        _body,
        grid=grid,
        in_specs=in_specs,
        out_specs=pl.BlockSpec((_G, 1), lambda i: (i, 0)),
        out_shape=jax.ShapeDtypeStruct((_B, 1), jnp.float32),
        compiler_params=pltpu.CompilerParams(
            dimension_semantics=("parallel",)),
    )(x3, cond, *leaves)
    return out.reshape(_B)


# speedup vs baseline: 1.4258x; 1.1005x over previous
import jax
import jax.numpy as jnp
from jax import lax
from jax.experimental import pallas as pl
from jax.experimental.pallas import tpu as pltpu

_B = 1024
_P = 100
_NF = 3
_NC = 2
_GNN = 30
_RD = 4
_K = 8
_G = 32
_PAD = 128


def _lrelu(a):
    return jnp.maximum(a, 0.2 * a)


def _dot(a, b):
    return jnp.dot(a, b, preferred_element_type=jnp.float32)


def _bdot(a, b, ca, cb):
    return lax.dot_general(
        a, b, (((ca,), (cb,)), ((0,), (0,))),
        preferred_element_type=jnp.float32)


def _body(x_ref, cond_ref, *refs):
    out_ref = refs[-1]
    (fW0, fb0, fW1, fb1, fW2, fb2,
     rW0, rb0, rW1, rb1,
     lW0, lb0, lW1, lb1,
     g1W0, g1b0, g1W1, g1b1,
     m1W0, m1b0, m1W1, m1b1,
     g2W0, g2b0, g2W1, g2b1,
     m2W0, m2b0, m2W1, m2b1,
     oW0, ob0, oW1, ob1) = [r[...] for r in refs[:-1]]

    x2 = x_ref[...].reshape(_G * _P, _NF)
    cond = cond_ref[...]

    f = _lrelu(_dot(x2, fW0) + fb0)
    f = _lrelu(_dot(f, fW1) + fb1)
    f = _lrelu(_dot(f, fW2) + fb2)
    pooled = jnp.max(f.reshape(_G, _P, 64), axis=1)
    r = _lrelu(_dot(pooled, rW0) + rb0)
    r = _dot(r, rW1) + rb1

    condb = jnp.broadcast_to(cond[:, None, :], (_G, _P, _NC))
    rb = jnp.broadcast_to(r[:, None, :], (_G, _P, _RD))
    cat = jnp.concatenate(
        [x_ref[...], condb, rb], axis=2).reshape(_G * _P, _NF + _NC + _RD)
    h = _lrelu(_dot(cat, lW0) + lb0)
    h = _dot(h, lW1) + lb1
    hb = h.reshape(_G, _P, _GNN)

    hbp = jnp.concatenate(
        [hb, jnp.zeros((_G, _PAD - _P, _GNN), jnp.float32)], axis=1)
    sq = jnp.sum(hb * hb, axis=-1)
    sqp = jnp.concatenate(
        [sq, jnp.full((_G, _PAD - _P), jnp.inf, jnp.float32)], axis=1)
    gram = _bdot(hb, hbp, 2, 2)
    d2 = sq[:, :, None] + sqp[:, None, :] - 2.0 * gram
    lane = lax.broadcasted_iota(
        jnp.int32, (_P, _PAD), 1).astype(jnp.float32)
    for _ in range(_K):
        m = jnp.min(d2, axis=2, keepdims=True)
        cand = jnp.where(d2 == m, lane, 128.0)
        jmin = jnp.min(cand, axis=2, keepdims=True)
        d2 = jnp.where(cand == jmin, jnp.inf, d2)
    adj = jnp.isinf(d2).astype(jnp.float32)

    aggr = _bdot(adj, hbp, 2, 1)
    z = (hb + aggr).reshape(_G * _P, _GNN)
    m = _lrelu(_dot(z, g1W0) + g1b0)
    m = _dot(m, g1W1) + g1b1
    mp = jnp.sum(m.reshape(_G, _P, _GNN), axis=1)
    pg = _dot(mp, m1W0[_GNN:]) + m1b0
    t = _dot(h, m1W0[:_GNN]).reshape(_G, _P, _GNN) + pg[:, None, :]
    h = _dot(_lrelu(t).reshape(_G * _P, _GNN), m1W1) + m1b1
    hb = h.reshape(_G, _P, _GNN)

    hbp = jnp.concatenate(
        [hb, jnp.zeros((_G, _PAD - _P, _GNN), jnp.float32)], axis=1)
    aggr = _bdot(adj, hbp, 2, 1)
    z = (hb + aggr).reshape(_G * _P, _GNN)
    m = _lrelu(_dot(z, g2W0) + g2b0)
    m = _dot(m, g2W1) + g2b1
    mp = jnp.sum(m.reshape(_G, _P, _GNN), axis=1)
    pg = _dot(mp, m2W0[_GNN:]) + m2b0
    t = _dot(h, m2W0[:_GNN]).reshape(_G, _P, _GNN) + pg[:, None, :]
    h = _dot(_lrelu(t).reshape(_G * _P, _GNN), m2W1) + m2b1

    ggn = jnp.sum(h.reshape(_G, _P, _GNN), axis=1)
    o = (_dot(r, oW0[:_RD]) + _dot(cond, oW0[_RD:_RD + _NC])
         + _dot(ggn, oW0[_RD + _NC:]) + ob0)
    o = _lrelu(o)
    o = _lrelu(_dot(o, oW1) + ob1)
    out_ref[...] = o


def kernel(x, cond, params, batch):
    del batch
    x3 = x.reshape(_B, _P, _NF)
    leaves = []
    for name in ('fc', 'rgan', 'lin', 'gin1', 'mpd1', 'gin2', 'mpd2', 'final'):
        for W, b in params[name]:
            leaves.append(W)
            leaves.append(b.reshape(1, -1))

    grid = (_B // _G,)
    in_specs = [
        pl.BlockSpec((_G, _P, _NF), lambda i: (i, 0, 0)),
        pl.BlockSpec((_G, _NC), lambda i: (i, 0)),
    ]
    for leaf in leaves:
        in_specs.append(pl.BlockSpec(leaf.shape, lambda i: (0,) * leaf.ndim))

    out = pl.pallas_call(
        _body,
        grid=grid,
        in_specs=in_specs,
        out_specs=pl.BlockSpec((_G, 1), lambda i: (i, 0)),
        out_shape=jax.ShapeDtypeStruct((_B, 1), jnp.float32),
        compiler_params=pltpu.CompilerParams(
            dimension_semantics=("parallel",)),
    )(x3, cond, *leaves)
    return out.reshape(_B)


# speedup vs baseline: 1.4918x; 1.0463x over previous
import jax
import jax.numpy as jnp
from jax import lax
from jax.experimental import pallas as pl
from jax.experimental.pallas import tpu as pltpu

_B = 1024
_P = 100
_NF = 3
_NC = 2
_GNN = 30
_RD = 4
_K = 8
_G = 64
_PAD = 128


def _lrelu(a):
    return jnp.maximum(a, 0.2 * a)


def _dot(a, b):
    return jnp.dot(a, b, preferred_element_type=jnp.float32)


def _bdot(a, b, ca, cb):
    return lax.dot_general(
        a, b, (((ca,), (cb,)), ((0,), (0,))),
        preferred_element_type=jnp.float32)


def _body(x_ref, cond_ref, *refs):
    out_ref = refs[-1]
    (fW0, fb0, fW1, fb1, fW2, fb2,
     rW0, rb0, rW1, rb1,
     lW0, lb0, lW1, lb1,
     g1W0, g1b0, g1W1, g1b1,
     m1W0, m1b0, m1W1, m1b1,
     g2W0, g2b0, g2W1, g2b1,
     m2W0, m2b0, m2W1, m2b1,
     oW0, ob0, oW1, ob1) = [r[...] for r in refs[:-1]]

    x2 = x_ref[...].reshape(_G * _P, _NF)
    cond = cond_ref[...]

    f = _lrelu(_dot(x2, fW0) + fb0)
    f = _lrelu(_dot(f, fW1) + fb1)
    f = _lrelu(_dot(f, fW2) + fb2)
    pooled = jnp.max(f.reshape(_G, _P, 64), axis=1)
    r = _lrelu(_dot(pooled, rW0) + rb0)
    r = _dot(r, rW1) + rb1

    condb = jnp.broadcast_to(cond[:, None, :], (_G, _P, _NC))
    rb = jnp.broadcast_to(r[:, None, :], (_G, _P, _RD))
    cat = jnp.concatenate(
        [x_ref[...], condb, rb], axis=2).reshape(_G * _P, _NF + _NC + _RD)
    h = _lrelu(_dot(cat, lW0) + lb0)
    h = _dot(h, lW1) + lb1
    hb = h.reshape(_G, _P, _GNN)

    hbp = jnp.concatenate(
        [hb, jnp.zeros((_G, _PAD - _P, _GNN), jnp.float32)], axis=1)
    sq = jnp.sum(hb * hb, axis=-1)
    sqp = jnp.concatenate(
        [sq, jnp.full((_G, _PAD - _P), jnp.inf, jnp.float32)], axis=1)
    gram = _bdot(hb, hbp, 2, 2)
    d2 = sq[:, :, None] + sqp[:, None, :] - 2.0 * gram
    lane = lax.broadcasted_iota(
        jnp.int32, (_P, _PAD), 1).astype(jnp.float32)
    for _ in range(_K):
        m = jnp.min(d2, axis=2, keepdims=True)
        cand = jnp.where(d2 == m, lane, 128.0)
        jmin = jnp.min(cand, axis=2, keepdims=True)
        d2 = jnp.where(cand == jmin, jnp.inf, d2)
    adj = jnp.isinf(d2).astype(jnp.float32)

    aggr = _bdot(adj, hbp, 2, 1)
    z = (hb + aggr).reshape(_G * _P, _GNN)
    m = _lrelu(_dot(z, g1W0) + g1b0)
    m = _dot(m, g1W1) + g1b1
    mp = jnp.sum(m.reshape(_G, _P, _GNN), axis=1)
    pg = _dot(mp, m1W0[_GNN:]) + m1b0
    t = _dot(h, m1W0[:_GNN]).reshape(_G, _P, _GNN) + pg[:, None, :]
    h = _dot(_lrelu(t).reshape(_G * _P, _GNN), m1W1) + m1b1
    hb = h.reshape(_G, _P, _GNN)

    hbp = jnp.concatenate(
        [hb, jnp.zeros((_G, _PAD - _P, _GNN), jnp.float32)], axis=1)
    aggr = _bdot(adj, hbp, 2, 1)
    z = (hb + aggr).reshape(_G * _P, _GNN)
    m = _lrelu(_dot(z, g2W0) + g2b0)
    m = _dot(m, g2W1) + g2b1
    mp = jnp.sum(m.reshape(_G, _P, _GNN), axis=1)
    pg = _dot(mp, m2W0[_GNN:]) + m2b0
    t = _dot(h, m2W0[:_GNN]).reshape(_G, _P, _GNN) + pg[:, None, :]
    h = _dot(_lrelu(t).reshape(_G * _P, _GNN), m2W1) + m2b1

    ggn = jnp.sum(h.reshape(_G, _P, _GNN), axis=1)
    o = (_dot(r, oW0[:_RD]) + _dot(cond, oW0[_RD:_RD + _NC])
         + _dot(ggn, oW0[_RD + _NC:]) + ob0)
    o = _lrelu(o)
    o = _lrelu(_dot(o, oW1) + ob1)
    out_ref[...] = o


def kernel(x, cond, params, batch):
    del batch
    x3 = x.reshape(_B, _P, _NF)
    leaves = []
    for name in ('fc', 'rgan', 'lin', 'gin1', 'mpd1', 'gin2', 'mpd2', 'final'):
        for W, b in params[name]:
            leaves.append(W)
            leaves.append(b.reshape(1, -1))

    grid = (_B // _G,)
    in_specs = [
        pl.BlockSpec((_G, _P, _NF), lambda i: (i, 0, 0)),
        pl.BlockSpec((_G, _NC), lambda i: (i, 0)),
    ]
    for leaf in leaves:
        in_specs.append(pl.BlockSpec(leaf.shape, lambda i: (0,) * leaf.ndim))

    out = pl.pallas_call(
        _body,
        grid=grid,
        in_specs=in_specs,
        out_specs=pl.BlockSpec((_G, 1), lambda i: (i, 0)),
        out_shape=jax.ShapeDtypeStruct((_B, 1), jnp.float32),
        compiler_params=pltpu.CompilerParams(
            dimension_semantics=("parallel",)),
    )(x3, cond, *leaves)
    return out.reshape(_B)
